# baseline (speedup 1.0000x reference)
import functools

import jax
import jax.numpy as jnp
from jax import lax
from jax.experimental import pallas as pl
from jax.experimental.pallas import tpu as pltpu
from jax.experimental.pallas import tpu_sc as plsc

_N_CODES = 20
_REPEAT = 10
_ROWS_PER_WORKER = 8


def _sc_info():
    info = plsc.get_sparse_core_info()
    return info.num_cores, info.num_subcores


def _make_sc_gather(total, dim):
    nc, ns = _sc_info()
    n_workers = (total + _ROWS_PER_WORKER - 1) // _ROWS_PER_WORKER
    assert n_workers <= nc * ns
    mesh = plsc.VectorSubcoreMesh(core_axis_name="c", subcore_axis_name="s")

    @functools.partial(
        pl.kernel,
        mesh=mesh,
        out_type=jax.ShapeDtypeStruct((total, dim), jnp.float32),
        scratch_types=[
            pltpu.VMEM((_ROWS_PER_WORKER,), jnp.int32),
            pltpu.VMEM((_ROWS_PER_WORKER, dim), jnp.float32),
            pltpu.SemaphoreType.DMA,
        ],
    )
    def sc_gather(w_hbm, seg_hbm, out_hbm, idx_v, rows_v, sem):
        wid = lax.axis_index("s") * nc + lax.axis_index("c")

        @pl.when(wid < n_workers)
        def _():
            base = wid * _ROWS_PER_WORKER
            pltpu.sync_copy(seg_hbm.at[pl.ds(base, _ROWS_PER_WORKER)], idx_v)
            pltpu.async_copy(w_hbm.at[idx_v], rows_v, sem).wait()
            pltpu.sync_copy(rows_v, out_hbm.at[pl.ds(base, _ROWS_PER_WORKER)])

    return sc_gather


def _tc_body(x_ref, wexp_ref, o_ref):
    i = pl.program_id(0)
    wexp_t = jnp.swapaxes(wexp_ref[:, :64], 0, 1)
    mask = (jax.lax.broadcasted_iota(jnp.int32, wexp_t.shape, 1) == i * _REPEAT)
    wcol = jnp.sum(jnp.where(mask, wexp_t, 0.0), axis=1, keepdims=True)
    o_ref[...] = x_ref[...] + wcol[None, :, :]


def kernel(X, W):
    B, T, D = X.shape
    seg = jnp.arange(T, dtype=jnp.int32) // _REPEAT
    w_pad = jnp.pad(W, ((0, 0), (0, 128 - D)))
    wexp = _make_sc_gather(T, 128)(w_pad, seg)
    Xt = jnp.transpose(X, (1, 2, 0))
    out_t = pl.pallas_call(
        _tc_body,
        grid=(_N_CODES,),
        in_specs=[
            pl.BlockSpec((_REPEAT, D, B), lambda i: (i, 0, 0)),
            pl.BlockSpec((T, 128), lambda i: (0, 0)),
        ],
        out_specs=pl.BlockSpec((_REPEAT, D, B), lambda i: (i, 0, 0)),
        out_shape=jax.ShapeDtypeStruct((T, D, B), X.dtype),
    )(Xt, wexp)
    return jnp.transpose(out_t, (2, 0, 1))


# speedup vs baseline: 1.1457x; 1.1457x over previous
import jax
import jax.numpy as jnp
from jax.experimental import pallas as pl

_N_CODES = 20
_REPEAT = 10


def _body(x_ref, wt_ref, o_ref):
    i = pl.program_id(0)
    wt = wt_ref[...]
    mask = (jax.lax.broadcasted_iota(jnp.int32, wt.shape, 1) == i)
    wcol = jnp.sum(jnp.where(mask, wt, 0.0), axis=1, keepdims=True)
    o_ref[...] = x_ref[...] + wcol[None, :, :]


def kernel(X, W):
    B, T, D = X.shape
    Xt = jnp.transpose(X, (1, 2, 0))
    Wt = jnp.transpose(W)
    out_t = pl.pallas_call(
        _body,
        grid=(_N_CODES,),
        in_specs=[
            pl.BlockSpec((_REPEAT, D, B), lambda i: (i, 0, 0)),
            pl.BlockSpec((D, _N_CODES), lambda i: (0, 0)),
        ],
        out_specs=pl.BlockSpec((_REPEAT, D, B), lambda i: (i, 0, 0)),
        out_shape=jax.ShapeDtypeStruct((T, D, B), X.dtype),
    )(Xt, Wt)
    return jnp.transpose(out_t, (2, 0, 1))
